# baseline (speedup 1.0000x reference)
import jax
import jax.numpy as jnp
from jax import lax
from jax.experimental import pallas as pl
from jax.experimental.pallas import tpu as pltpu

B, S, H = 1, 2048, 1024
I = 512
E = 8
NGROUP = 4
GSIZE = E // NGROUP
SCALE = 1.0

TS = 512
NT = S // TS


def _gate(x, gw, bias):
    logits = jax.lax.dot_general(
        x, gw, (((1,), (1,)), ((), ())),
        preferred_element_type=jnp.float32)
    scores = jax.nn.sigmoid(logits)
    s4c = scores + bias

    gcols = [s4c[:, 2 * g:2 * g + 1] + s4c[:, 2 * g + 1:2 * g + 2]
             for g in range(NGROUP)]
    gexp = jnp.concatenate(
        [gcols[g] for g in range(NGROUP) for _ in range(GSIZE)], axis=1)

    eids = lax.broadcasted_iota(jnp.int32, (1, E), 1)
    gids = eids // GSIZE

    cnt = jnp.zeros((S, E), jnp.int32)
    for gp in range(NGROUP):
        gsp = gcols[gp]
        beats = (gsp > gexp) | ((gsp == gexp) & (gp < gids))
        cnt = cnt + beats.astype(jnp.int32)
    gmask = cnt < 2

    tmp = jnp.where(gmask, s4c, 0.0)
    cnt2 = jnp.zeros((S, E), jnp.int32)
    for ep in range(E):
        v = tmp[:, ep:ep + 1]
        beats = (v > tmp) | ((v == tmp) & (ep < eids))
        cnt2 = cnt2 + beats.astype(jnp.int32)
    sel = cnt2 < 2

    w = jnp.where(sel, scores, 0.0)
    denom = jnp.sum(w, axis=1, keepdims=True) + 1e-20
    return w / denom * SCALE


def _fused_body(x_ref, gw_ref, b_ref, gp_ref, up_ref, dp_ref, out_ref,
                wg_s, wu_s, wd_s, comb_s, acc_s):
    e = pl.program_id(0)
    t = pl.program_id(1)

    @pl.when((e == 0) & (t == 0))
    def _gate_step():
        comb_s[...] = _gate(x_ref[...], gw_ref[...], b_ref[...])

    @pl.when(t == 0)
    def _cast():
        wg_s[...] = gp_ref[0].astype(jnp.bfloat16)
        wu_s[...] = up_ref[0].astype(jnp.bfloat16)
        wd_s[...] = dp_ref[0].astype(jnp.bfloat16)

    row = pl.ds(t * TS, TS)
    x = x_ref[row, :].astype(jnp.bfloat16)
    comb = comb_s[row, :]
    lane = lax.broadcasted_iota(jnp.int32, (1, E), 1)
    w = jnp.sum(jnp.where(lane == e, comb, 0.0), axis=1, keepdims=True)
    g = jax.lax.dot_general(x, wg_s[...], (((1,), (1,)), ((), ())),
                            preferred_element_type=jnp.float32)
    u = jax.lax.dot_general(x, wu_s[...], (((1,), (1,)), ((), ())),
                            preferred_element_type=jnp.float32)
    hact = (g * jax.nn.sigmoid(g) * u).astype(jnp.bfloat16)
    y = jax.lax.dot_general(hact, wd_s[...], (((1,), (1,)), ((), ())),
                            preferred_element_type=jnp.float32)
    contrib = w * y

    @pl.when(e == 0)
    def _init():
        acc_s[row, :] = contrib

    @pl.when((e > 0) & (e < E - 1))
    def _accum():
        acc_s[row, :] += contrib

    @pl.when(e == E - 1)
    def _final():
        out_ref[...] = acc_s[row, :] + contrib


@jax.jit
def _run(x, gate_weight, bias2d, gate_proj, up_proj, down_proj):
    out = pl.pallas_call(
        _fused_body,
        grid=(E, NT),
        in_specs=[
            pl.BlockSpec((S, H), lambda e, t: (0, 0)),
            pl.BlockSpec((E, H), lambda e, t: (0, 0)),
            pl.BlockSpec((1, E), lambda e, t: (0, 0)),
            pl.BlockSpec((1, I, H), lambda e, t: (e, 0, 0)),
            pl.BlockSpec((1, I, H), lambda e, t: (e, 0, 0)),
            pl.BlockSpec((1, H, I), lambda e, t: (e, 0, 0)),
        ],
        out_specs=pl.BlockSpec(
            (TS, H), lambda e, t: (jnp.where(e == E - 1, t, 0), 0)),
        out_shape=jax.ShapeDtypeStruct((S, H), jnp.float32),
        scratch_shapes=[
            pltpu.VMEM((I, H), jnp.bfloat16),
            pltpu.VMEM((I, H), jnp.bfloat16),
            pltpu.VMEM((H, I), jnp.bfloat16),
            pltpu.VMEM((S, E), jnp.float32),
            pltpu.VMEM((S, H), jnp.float32),
        ],
    )(x, gate_weight, bias2d, gate_proj, up_proj, down_proj)
    return out


def kernel(hidden_states, gate_weight, e_score_correction_bias,
           gate_proj, up_proj, down_proj):
    x = hidden_states.reshape(-1, H).astype(jnp.float32)
    bias2d = e_score_correction_bias.reshape(1, E).astype(jnp.float32)
    out = _run(x, gate_weight, bias2d, gate_proj, up_proj, down_proj)
    return out.reshape(hidden_states.shape)


# speedup vs baseline: 1.0857x; 1.0857x over previous
import jax
import jax.numpy as jnp
from jax import lax
from jax.experimental import pallas as pl
from jax.experimental.pallas import tpu as pltpu

B, S, H = 1, 2048, 1024
I = 512
E = 8
NGROUP = 4
GSIZE = E // NGROUP
SCALE = 1.0

TS = 512
NT = S // TS


def _gate(x, gw, bias):
    logits = jax.lax.dot_general(
        x, gw, (((1,), (1,)), ((), ())),
        preferred_element_type=jnp.float32)
    scores = jax.nn.sigmoid(logits)
    s4c = scores + bias

    gcols = [s4c[:, 2 * g:2 * g + 1] + s4c[:, 2 * g + 1:2 * g + 2]
             for g in range(NGROUP)]
    gexp = jnp.concatenate(
        [gcols[g] for g in range(NGROUP) for _ in range(GSIZE)], axis=1)

    eids = lax.broadcasted_iota(jnp.int32, (1, E), 1)
    gids = eids // GSIZE

    cnt = jnp.zeros((S, E), jnp.int32)
    for gp in range(NGROUP):
        gsp = gcols[gp]
        beats = (gsp > gexp) | ((gsp == gexp) & (gp < gids))
        cnt = cnt + beats.astype(jnp.int32)
    gmask = cnt < 2

    tmp = jnp.where(gmask, s4c, 0.0)
    cnt2 = jnp.zeros((S, E), jnp.int32)
    for ep in range(E):
        v = tmp[:, ep:ep + 1]
        beats = (v > tmp) | ((v == tmp) & (ep < eids))
        cnt2 = cnt2 + beats.astype(jnp.int32)
    sel = cnt2 < 2

    w = jnp.where(sel, scores, 0.0)
    denom = jnp.sum(w, axis=1, keepdims=True) + 1e-20
    return w / denom * SCALE


def _fused_body(x_ref, gw_ref, b_ref, gp_ref, up_ref, dp_ref, out_ref,
                wg_s, wu_s, wd_s, comb_s):
    s = pl.program_id(0)

    @pl.when(s < E)
    def _cast():
        wg_s[pl.ds(s, 1)] = gp_ref[...].astype(jnp.bfloat16)
        wu_s[pl.ds(s, 1)] = up_ref[...].astype(jnp.bfloat16)
        wd_s[pl.ds(s, 1)] = dp_ref[...].astype(jnp.bfloat16)

    @pl.when(s == 0)
    def _gate_step():
        comb_s[...] = _gate(x_ref[...], gw_ref[...], b_ref[...])

    @pl.when(s >= E)
    def _compute():
        t = s - E
        row = pl.ds(t * TS, TS)
        x = x_ref[row, :].astype(jnp.bfloat16)
        comb = comb_s[row, :]
        acc = jnp.zeros((TS, H), jnp.float32)
        lane = lax.broadcasted_iota(jnp.int32, (1, E), 1)
        for e in range(E):
            w = jnp.sum(jnp.where(lane == e, comb, 0.0), axis=1,
                        keepdims=True)
            g = jax.lax.dot_general(x, wg_s[e], (((1,), (1,)), ((), ())),
                                    preferred_element_type=jnp.float32)
            u = jax.lax.dot_general(x, wu_s[e], (((1,), (1,)), ((), ())),
                                    preferred_element_type=jnp.float32)
            hact = (g * jax.nn.sigmoid(g) * u).astype(jnp.bfloat16)
            y = jax.lax.dot_general(hact, wd_s[e], (((1,), (1,)), ((), ())),
                                    preferred_element_type=jnp.float32)
            acc = acc + w * y
        out_ref[...] = acc


@jax.jit
def _run(x, gate_weight, bias2d, gate_proj, up_proj, down_proj):
    out = pl.pallas_call(
        _fused_body,
        grid=(E + NT,),
        in_specs=[
            pl.BlockSpec((S, H), lambda s: (0, 0)),
            pl.BlockSpec((E, H), lambda s: (0, 0)),
            pl.BlockSpec((1, E), lambda s: (0, 0)),
            pl.BlockSpec((1, I, H), lambda s: (jnp.minimum(s, E - 1), 0, 0)),
            pl.BlockSpec((1, I, H), lambda s: (jnp.minimum(s, E - 1), 0, 0)),
            pl.BlockSpec((1, H, I), lambda s: (jnp.minimum(s, E - 1), 0, 0)),
        ],
        out_specs=pl.BlockSpec((TS, H), lambda s: (jnp.maximum(s - E, 0), 0)),
        out_shape=jax.ShapeDtypeStruct((S, H), jnp.float32),
        scratch_shapes=[
            pltpu.VMEM((E, I, H), jnp.bfloat16),
            pltpu.VMEM((E, I, H), jnp.bfloat16),
            pltpu.VMEM((E, H, I), jnp.bfloat16),
            pltpu.VMEM((S, E), jnp.float32),
        ],
    )(x, gate_weight, bias2d, gate_proj, up_proj, down_proj)
    return out


def kernel(hidden_states, gate_weight, e_score_correction_bias,
           gate_proj, up_proj, down_proj):
    x = hidden_states.reshape(-1, H).astype(jnp.float32)
    bias2d = e_score_correction_bias.reshape(1, E).astype(jnp.float32)
    out = _run(x, gate_weight, bias2d, gate_proj, up_proj, down_proj)
    return out.reshape(hidden_states.shape)
